# baseline (speedup 1.0000x reference)
import functools

import jax
import jax.numpy as jnp
from jax import lax
from jax.experimental import pallas as pl
from jax.experimental.pallas import tpu as pltpu
from jax.experimental.pallas import tpu_sc as plsc

EMBEDDING_DIM = 64
RANK = 8
LORA_SCALING = 16.0 / 8.0

NUM_CORES = 2
NUM_SUBCORES = 16
NUM_WORKERS = NUM_CORES * NUM_SUBCORES
CHUNK = 128
NBUF = 2
LANES = 16
DCH = EMBEDDING_DIM // LANES


def _full16(v):
    return jnp.full((LANES,), v, jnp.int32)


@functools.partial(jax.jit, static_argnames=("n_chunks",))
def _lora_lookup(xf, weight, lora_A, lora_B, n_chunks):
    n = NUM_WORKERS * n_chunks * CHUNK

    def body(x_hbm, w_hbm, a_hbm, b_hbm, out_hbm, idx_v, b_v,
             w0, w1, a0, a1, o0, o1, wsem0, wsem1, asem0, asem1, osem0, osem1):
        wid = lax.axis_index("s") * NUM_CORES + lax.axis_index("c")
        base_row = wid * (n_chunks * CHUNK)

        w_bufs = (w0, w1)
        a_bufs = (a0, a1)
        o_bufs = (o0, o1)
        wsems = (wsem0, wsem1)
        asems = (asem0, asem1)
        osems = (osem0, osem1)

        pltpu.sync_copy(x_hbm.at[wid], idx_v)
        pltpu.sync_copy(b_hbm, b_v)

        bs = [[b_v[r, pl.ds(c * LANES, LANES)] * LORA_SCALING for c in range(DCH)]
              for r in range(RANK)]

        for b in range(NBUF):
            pltpu.make_async_copy(w_hbm.at[idx_v.at[b]], w_bufs[b], wsems[b]).start()
            pltpu.make_async_copy(a_hbm.at[idx_v.at[b]], a_bufs[b], asems[b]).start()

        @pl.loop(0, n_chunks, step=NBUF)
        def chunk_loop(g0):
            for b in range(NBUF):
                g = g0 + b
                pltpu.make_async_copy(w_hbm.at[idx_v.at[g]], w_bufs[b], wsems[b]).wait()
                pltpu.make_async_copy(a_hbm.at[idx_v.at[g]], a_bufs[b], asems[b]).wait()

                @pl.when(g0 > 0)
                def _():
                    pltpu.make_async_copy(
                        o_bufs[b],
                        out_hbm.at[pl.ds(base_row + (g - NBUF) * CHUNK, CHUNK)],
                        osems[b]).wait()

                w_b, a_b, o_b = w_bufs[b], a_bufs[b], o_bufs[b]

                @pl.loop(0, CHUNK)
                def row_loop(i):
                    ii = _full16(i)
                    ab = [plsc.load_gather(a_b, [ii, _full16(r)]) for r in range(RANK)]
                    for c in range(DCH):
                        w = w_b[i, pl.ds(c * LANES, LANES)]
                        t0 = ab[0] * bs[0][c] + ab[1] * bs[1][c]
                        t1 = ab[2] * bs[2][c] + ab[3] * bs[3][c]
                        t2 = ab[4] * bs[4][c] + ab[5] * bs[5][c]
                        t3 = ab[6] * bs[6][c] + ab[7] * bs[7][c]
                        o_b[i, pl.ds(c * LANES, LANES)] = w + ((t0 + t1) + (t2 + t3))

                pltpu.make_async_copy(
                    o_b, out_hbm.at[pl.ds(base_row + g * CHUNK, CHUNK)],
                    osems[b]).start()

                @pl.when(g + NBUF < n_chunks)
                def _():
                    pltpu.make_async_copy(
                        w_hbm.at[idx_v.at[g + NBUF]], w_bufs[b], wsems[b]).start()
                    pltpu.make_async_copy(
                        a_hbm.at[idx_v.at[g + NBUF]], a_bufs[b], asems[b]).start()

        for b in range(NBUF):
            g = n_chunks - NBUF + b
            pltpu.make_async_copy(
                o_bufs[b], out_hbm.at[pl.ds(base_row + g * CHUNK, CHUNK)],
                osems[b]).wait()

    run = pl.kernel(
        body,
        out_type=jax.ShapeDtypeStruct((n, EMBEDDING_DIM), jnp.float32),
        mesh=plsc.VectorSubcoreMesh(core_axis_name="c", subcore_axis_name="s"),
        compiler_params=pltpu.CompilerParams(
            needs_layout_passes=False, use_tc_tiling_on_sc=False),
        scratch_types=[
            pltpu.VMEM((n_chunks, CHUNK), jnp.int32),
            pltpu.VMEM((RANK, EMBEDDING_DIM), jnp.float32),
            pltpu.VMEM((CHUNK, EMBEDDING_DIM), jnp.float32),
            pltpu.VMEM((CHUNK, EMBEDDING_DIM), jnp.float32),
            pltpu.VMEM((CHUNK, RANK), jnp.float32),
            pltpu.VMEM((CHUNK, RANK), jnp.float32),
            pltpu.VMEM((CHUNK, EMBEDDING_DIM), jnp.float32),
            pltpu.VMEM((CHUNK, EMBEDDING_DIM), jnp.float32),
            pltpu.SemaphoreType.DMA,
            pltpu.SemaphoreType.DMA,
            pltpu.SemaphoreType.DMA,
            pltpu.SemaphoreType.DMA,
            pltpu.SemaphoreType.DMA,
            pltpu.SemaphoreType.DMA,
        ],
    )
    return run(xf, weight, lora_A, lora_B)


def kernel(x, weight, lora_A, lora_B):
    batch_shape = x.shape
    n = x.size
    assert n % (NUM_WORKERS * CHUNK) == 0
    n_chunks = n // (NUM_WORKERS * CHUNK)
    xf = x.reshape(NUM_WORKERS, n_chunks, CHUNK).astype(jnp.int32)
    out = _lora_lookup(xf, weight, lora_A, lora_B, n_chunks)
    return out.reshape(*batch_shape, EMBEDDING_DIM)
